# baseline (speedup 1.0000x reference)
import functools

import jax
import jax.numpy as jnp
from jax import lax
from jax.experimental import pallas as pl
from jax.experimental.pallas import tpu as pltpu
from jax.experimental.pallas import tpu_sc as plsc

NUM_SEQ = 16384
T = 50
D = 64
H = 8
B = 4096
S = H * H
ROW = T * D

NC = 2
NS = 16
NW = NC * NS
B_PER_W = B // NW
CHUNK = 32

NB = 512
NBLK = B // NB


def _make_sc_gather():
    mesh = plsc.VectorSubcoreMesh(core_axis_name="c", subcore_axis_name="s")

    @functools.partial(
        pl.kernel,
        mesh=mesh,
        out_type=[
            jax.ShapeDtypeStruct((B, ROW), jnp.float32),
            jax.ShapeDtypeStruct((B,), jnp.int32),
        ],
        scratch_types=[
            pltpu.VMEM((B_PER_W,), jnp.int32),
            pltpu.VMEM((CHUNK, ROW), jnp.float32),
            pltpu.VMEM((B_PER_W,), jnp.int32),
            pltpu.SemaphoreType.DMA,
            pltpu.SemaphoreType.DMA,
        ],
    )
    def gather_k(table_hbm, idx_hbm, lens_hbm, y_out, lens_out,
                 idx_v, rows_v, lens_loc, sem_r, sem_l):
        wid = lax.axis_index("s") * NC + lax.axis_index("c")
        base = wid * B_PER_W
        pltpu.sync_copy(idx_hbm.at[pl.ds(base, B_PER_W)], idx_v)
        cp_l = pltpu.async_copy(lens_hbm.at[idx_v], lens_loc, sem_l)
        cp_l.wait()
        pltpu.sync_copy(lens_loc, lens_out.at[pl.ds(base, B_PER_W)])
        for c in range(B_PER_W // CHUNK):
            off = base + c * CHUNK
            cp_r = pltpu.async_copy(
                table_hbm.at[idx_v.at[pl.ds(c * CHUNK, CHUNK)]], rows_v, sem_r)
            cp_r.wait()
            pltpu.sync_copy(rows_v, y_out.at[pl.ds(off, CHUNK)])

    return gather_k


CB = 64


def _fwd_body(y_ref, len_ref, K_ref, init_ref, ET_ref, bias_ref, out_ref,
              eb_ref, c_ref):
    ET = ET_ref[...]
    Km = K_ref[...]
    bias = bias_ref[...]
    lens = len_ref[...]

    for k in range(NB // CB):
        y2 = y_ref[pl.ds(k * CB, CB), :, :].reshape(CB * T, D)
        lb = jnp.dot(y2, ET, preferred_element_type=jnp.float32) + bias
        c = jnp.max(lb, axis=-1, keepdims=True)
        eb_ref[pl.ds(k * CB, CB)] = jnp.exp(lb - c).reshape(CB, T, S)
        c_ref[pl.ds(k * CB, CB)] = c.reshape(CB, T)

    tmask = jax.lax.broadcasted_iota(jnp.int32, (NB, T), 1) < lens
    ll = jnp.sum(jnp.where(tmask, c_ref[...], 0.0), axis=-1, keepdims=True)

    alpha = init_ref[...] * eb_ref[:, 0, :]
    for t in range(1, T):
        pred = jnp.dot(alpha, Km, preferred_element_type=jnp.float32)
        alpha = jnp.where(lens > t, pred * eb_ref[:, t, :], alpha)
        if t % 4 == 0:
            s = jnp.sum(alpha, axis=-1, keepdims=True)
            alpha = alpha / s
            ll = ll + jnp.log(s)
    s = jnp.sum(alpha, axis=-1, keepdims=True)
    out_ref[...] = ll + jnp.log(s)


def _make_tc_compute(interpret=False):
    return pl.pallas_call(
        _fwd_body,
        grid=(NBLK,),
        in_specs=[
            pl.BlockSpec((NB, T, D), lambda i: (i, 0, 0)),
            pl.BlockSpec((NB, 1), lambda i: (i, 0)),
            pl.BlockSpec((S, S), lambda i: (0, 0)),
            pl.BlockSpec((1, S), lambda i: (0, 0)),
            pl.BlockSpec((D, S), lambda i: (0, 0)),
            pl.BlockSpec((1, S), lambda i: (0, 0)),
        ],
        out_specs=pl.BlockSpec((NB, 1), lambda i: (i, 0)),
        out_shape=jax.ShapeDtypeStruct((B, 1), jnp.float32),
        scratch_shapes=[
            pltpu.VMEM((NB, T, S), jnp.float32),
            pltpu.VMEM((NB, T), jnp.float32),
        ],
        interpret=interpret,
    )


def kernel(sequences, lengths, mb, mask, probs_w, w_init, probs_x, x_init,
           probs_y):
    eps = 1e-6
    pw = probs_w + eps
    pw = pw / pw.sum(-1, keepdims=True)
    px = probs_x + eps
    px = px / px.sum(-1, keepdims=True)
    pwi = w_init + eps
    pwi = pwi / pwi.sum()
    pxi = x_init + eps
    pxi = pxi / pxi.sum()
    py = jnp.clip(probs_y, eps, 1.0 - eps)
    lpy = jnp.log(py)
    l1m = jnp.log1p(-py)
    ET = (lpy - l1m).reshape(S, D).T
    bias = l1m.sum(-1).reshape(1, S)
    Km = (pw[:, None, :, None] * px[None, :, None, :]).reshape(S, S)
    init = (pwi[:, None] * pxi[None, :]).reshape(1, S)

    table = sequences.reshape(NUM_SEQ, ROW)
    y_g, lens_g = _make_sc_gather()(table, mb.astype(jnp.int32),
                                    lengths.astype(jnp.int32))
    len_mb = lens_g[:, None]

    ll = _make_tc_compute()(
        y_g.reshape(B, T, D), len_mb, Km, init, ET, bias)
    return jnp.where(mask, ll[:, 0], 0.0)


# speedup vs baseline: 1.5455x; 1.5455x over previous
import functools

import jax
import jax.numpy as jnp
from jax import lax
from jax.experimental import pallas as pl
from jax.experimental.pallas import tpu as pltpu
from jax.experimental.pallas import tpu_sc as plsc

NUM_SEQ = 16384
T = 50
D = 64
H = 8
B = 4096
S = H * H
ROW = T * D

NC = 2
NS = 16
NW = NC * NS
B_PER_W = B // NW
CHUNK = 32

NB = 512
NBLK = B // NB


def _make_sc_gather():
    mesh = plsc.VectorSubcoreMesh(core_axis_name="c", subcore_axis_name="s")

    @functools.partial(
        pl.kernel,
        mesh=mesh,
        out_type=[
            jax.ShapeDtypeStruct((B, ROW), jnp.float32),
            jax.ShapeDtypeStruct((B,), jnp.int32),
        ],
        scratch_types=[
            pltpu.VMEM((B_PER_W,), jnp.int32),
            pltpu.VMEM((CHUNK, ROW), jnp.float32),
            pltpu.VMEM((B_PER_W,), jnp.int32),
            pltpu.SemaphoreType.DMA,
            pltpu.SemaphoreType.DMA,
        ],
    )
    def gather_k(table_hbm, idx_hbm, lens_hbm, y_out, lens_out,
                 idx_v, rows_v, lens_loc, sem_r, sem_l):
        wid = lax.axis_index("s") * NC + lax.axis_index("c")
        base = wid * B_PER_W
        pltpu.sync_copy(idx_hbm.at[pl.ds(base, B_PER_W)], idx_v)
        cp_l = pltpu.async_copy(lens_hbm.at[idx_v], lens_loc, sem_l)
        cp_l.wait()
        pltpu.sync_copy(lens_loc, lens_out.at[pl.ds(base, B_PER_W)])
        for c in range(B_PER_W // CHUNK):
            off = base + c * CHUNK
            cp_r = pltpu.async_copy(
                table_hbm.at[idx_v.at[pl.ds(c * CHUNK, CHUNK)]], rows_v, sem_r)
            cp_r.wait()
            pltpu.sync_copy(rows_v, y_out.at[pl.ds(off, CHUNK)])

    return gather_k


def _fwd_body(y_ref, len_ref, K_ref, init_ref, ET_ref, bias_ref, out_ref):
    ET = ET_ref[...]
    Km = K_ref[...]
    bias = bias_ref[...]
    lens = len_ref[...]

    lb = jnp.dot(y_ref[:, 0:D], ET,
                 preferred_element_type=jnp.float32) + bias
    c = jnp.max(lb, axis=-1, keepdims=True)
    alpha = init_ref[...] * jnp.exp(lb - c)
    ll = c
    for t in range(1, T):
        lb = jnp.dot(y_ref[:, t * D:(t + 1) * D], ET,
                     preferred_element_type=jnp.float32) + bias
        c = jnp.max(lb, axis=-1, keepdims=True)
        eb = jnp.exp(lb - c)
        pred = jnp.dot(alpha, Km, preferred_element_type=jnp.float32)
        act = lens > t
        alpha = jnp.where(act, pred * eb, alpha)
        ll = jnp.where(act, ll + c, ll)
        if t % 4 == 0:
            s = jnp.sum(alpha, axis=-1, keepdims=True)
            alpha = alpha / s
            ll = ll + jnp.log(s)
    s = jnp.sum(alpha, axis=-1, keepdims=True)
    out_ref[...] = ll + jnp.log(s)


def _make_tc_compute(interpret=False):
    return pl.pallas_call(
        _fwd_body,
        grid=(NBLK,),
        in_specs=[
            pl.BlockSpec((NB, ROW), lambda i: (i, 0)),
            pl.BlockSpec((NB, 1), lambda i: (i, 0)),
            pl.BlockSpec((S, S), lambda i: (0, 0)),
            pl.BlockSpec((1, S), lambda i: (0, 0)),
            pl.BlockSpec((D, S), lambda i: (0, 0)),
            pl.BlockSpec((1, S), lambda i: (0, 0)),
        ],
        out_specs=pl.BlockSpec((NB, 1), lambda i: (i, 0)),
        out_shape=jax.ShapeDtypeStruct((B, 1), jnp.float32),
        interpret=interpret,
    )


def kernel(sequences, lengths, mb, mask, probs_w, w_init, probs_x, x_init,
           probs_y):
    eps = 1e-6
    pw = probs_w + eps
    pw = pw / pw.sum(-1, keepdims=True)
    px = probs_x + eps
    px = px / px.sum(-1, keepdims=True)
    pwi = w_init + eps
    pwi = pwi / pwi.sum()
    pxi = x_init + eps
    pxi = pxi / pxi.sum()
    py = jnp.clip(probs_y, eps, 1.0 - eps)
    lpy = jnp.log(py)
    l1m = jnp.log1p(-py)
    ET = (lpy - l1m).reshape(S, D).T
    bias = l1m.sum(-1).reshape(1, S)
    Km = (pw[:, None, :, None] * px[None, :, None, :]).reshape(S, S)
    init = (pwi[:, None] * pxi[None, :]).reshape(1, S)

    table = sequences.reshape(NUM_SEQ, ROW)
    y_g, lens_g = _make_sc_gather()(table, mb.astype(jnp.int32),
                                    lengths.astype(jnp.int32))
    len_mb = lens_g[:, None]

    ll = _make_tc_compute()(y_g, len_mb, Km, init, ET, bias)
    return jnp.where(mask, ll[:, 0], 0.0)


# speedup vs baseline: 1.7601x; 1.1389x over previous
import functools

import jax
import jax.numpy as jnp
from jax import lax
from jax.experimental import pallas as pl
from jax.experimental.pallas import tpu as pltpu
from jax.experimental.pallas import tpu_sc as plsc

NUM_SEQ = 16384
T = 50
D = 64
H = 8
B = 4096
S = H * H
ROW = T * D

NC = 2
NS = 16
NW = NC * NS
B_PER_W = B // NW
CHUNK = 32

NB = 1024
NBLK = B // NB


def _make_sc_gather():
    mesh = plsc.VectorSubcoreMesh(core_axis_name="c", subcore_axis_name="s")

    @functools.partial(
        pl.kernel,
        mesh=mesh,
        out_type=[
            jax.ShapeDtypeStruct((B, ROW), jnp.float32),
            jax.ShapeDtypeStruct((B,), jnp.int32),
        ],
        scratch_types=[
            pltpu.VMEM((B_PER_W,), jnp.int32),
            pltpu.VMEM((CHUNK, ROW), jnp.float32),
            pltpu.VMEM((B_PER_W,), jnp.int32),
            pltpu.SemaphoreType.DMA,
            pltpu.SemaphoreType.DMA,
        ],
    )
    def gather_k(table_hbm, idx_hbm, lens_hbm, y_out, lens_out,
                 idx_v, rows_v, lens_loc, sem_r, sem_l):
        wid = lax.axis_index("s") * NC + lax.axis_index("c")
        base = wid * B_PER_W
        pltpu.sync_copy(idx_hbm.at[pl.ds(base, B_PER_W)], idx_v)
        cp_l = pltpu.async_copy(lens_hbm.at[idx_v], lens_loc, sem_l)
        cp_l.wait()
        pltpu.sync_copy(lens_loc, lens_out.at[pl.ds(base, B_PER_W)])
        for c in range(B_PER_W // CHUNK):
            off = base + c * CHUNK
            cp_r = pltpu.async_copy(
                table_hbm.at[idx_v.at[pl.ds(c * CHUNK, CHUNK)]], rows_v, sem_r)
            cp_r.wait()
            pltpu.sync_copy(rows_v, y_out.at[pl.ds(off, CHUNK)])

    return gather_k


def _fwd_body(y_ref, len_ref, K_ref, init_ref, ET2_ref, bias2_ref, out_ref):
    ET2 = ET2_ref[...]
    Km = K_ref[...]
    bias2 = bias2_ref[...]
    lens = len_ref[...]

    def emit(p):
        yp = y_ref[:, 2 * p * D:(2 * p + 2) * D].astype(jnp.bfloat16)
        lb = jnp.dot(yp, ET2, preferred_element_type=jnp.float32) + bias2
        c = jnp.max(lb, axis=-1, keepdims=True)
        return c, jnp.exp(lb - c)

    def trans(alpha):
        return jnp.dot(alpha.astype(jnp.bfloat16), Km,
                       preferred_element_type=jnp.float32)

    c, eb = emit(0)
    alpha = init_ref[...] * eb[:, 0:S]
    ll = c
    act = lens > 1
    alpha = jnp.where(act, trans(alpha) * eb[:, S:2 * S], alpha)
    ll = jnp.where(act, ll + c, ll)
    for p in range(1, T // 2):
        c, eb = emit(p)
        act = lens > 2 * p
        alpha = jnp.where(act, trans(alpha) * eb[:, 0:S], alpha)
        ll = jnp.where(act, ll + c, ll)
        act = lens > 2 * p + 1
        alpha = jnp.where(act, trans(alpha) * eb[:, S:2 * S], alpha)
        ll = jnp.where(act, ll + c, ll)
        if p % 2 == 1:
            s = jnp.sum(alpha, axis=-1, keepdims=True)
            alpha = alpha / s
            ll = ll + jnp.log(s)
    s = jnp.sum(alpha, axis=-1, keepdims=True)
    out_ref[...] = ll + jnp.log(s)


def _make_tc_compute(interpret=False):
    return pl.pallas_call(
        _fwd_body,
        grid=(NBLK,),
        in_specs=[
            pl.BlockSpec((NB, ROW), lambda i: (i, 0)),
            pl.BlockSpec((NB, 1), lambda i: (i, 0)),
            pl.BlockSpec((S, S), lambda i: (0, 0)),
            pl.BlockSpec((1, S), lambda i: (0, 0)),
            pl.BlockSpec((2 * D, 2 * S), lambda i: (0, 0)),
            pl.BlockSpec((1, 2 * S), lambda i: (0, 0)),
        ],
        out_specs=pl.BlockSpec((NB, 1), lambda i: (i, 0)),
        out_shape=jax.ShapeDtypeStruct((B, 1), jnp.float32),
        interpret=interpret,
    )


def kernel(sequences, lengths, mb, mask, probs_w, w_init, probs_x, x_init,
           probs_y):
    eps = 1e-6
    pw = probs_w + eps
    pw = pw / pw.sum(-1, keepdims=True)
    px = probs_x + eps
    px = px / px.sum(-1, keepdims=True)
    pwi = w_init + eps
    pwi = pwi / pwi.sum()
    pxi = x_init + eps
    pxi = pxi / pxi.sum()
    py = jnp.clip(probs_y, eps, 1.0 - eps)
    lpy = jnp.log(py)
    l1m = jnp.log1p(-py)
    ET = (lpy - l1m).reshape(S, D).T
    ET2 = jnp.zeros((2 * D, 2 * S), jnp.float32)
    ET2 = ET2.at[:D, :S].set(ET).at[D:, S:].set(ET).astype(jnp.bfloat16)
    bias = l1m.sum(-1).reshape(1, S)
    bias2 = jnp.concatenate([bias, bias], axis=1)
    Km = (pw[:, None, :, None] * px[None, :, None, :]).reshape(S, S)
    Km = Km.astype(jnp.bfloat16)
    init = (pwi[:, None] * pxi[None, :]).reshape(1, S)

    table = sequences.reshape(NUM_SEQ, ROW)
    y_g, lens_g = _make_sc_gather()(table, mb.astype(jnp.int32),
                                    lengths.astype(jnp.int32))
    len_mb = lens_g[:, None]

    ll = _make_tc_compute()(y_g, len_mb, Km, init, ET2, bias2)
    return jnp.where(mask, ll[:, 0], 0.0)
